# baseline (speedup 1.0000x reference)
import functools

import jax
import jax.numpy as jnp
from jax import lax
from jax.experimental import pallas as pl
from jax.experimental.pallas import tpu as pltpu
from jax.experimental.pallas import tpu_sc as plsc

N = 10000
NFEAT = 128
NHID = 128
NH = N // 2

BM = 200
R_SC = 3840
N_TC = N - R_SC
BM2 = 80
RK_SC = 2048
NK_TC = NH - RK_SC
BK2 = 72

NW = 32
NCH = N // 16
UNROLL = 25
RG = 4

NB1 = N // BM
NB2 = N_TC // BM2
NBK = NK_TC // BK2


def _pass1_body(adj_ref, x_ref, w1_ref, b1_ref, w2_ref, v_ref, s1_ref):
    i = pl.program_id(0)

    @pl.when(i == 0)
    def _():
        s1_ref[...] = jnp.dot(x_ref[...], w1_ref[...],
                              preferred_element_type=jnp.float32)

    h = jnp.dot(adj_ref[...], s1_ref[...],
                preferred_element_type=jnp.float32)
    h = jnp.maximum(h + b1_ref[...], 0.0)
    v_ref[...] = jnp.dot(h, w2_ref[...],
                         preferred_element_type=jnp.float32)


_sc_mesh = plsc.VectorSubcoreMesh(core_axis_name="c", subcore_axis_name="s")


def _make_sc_matvec(row_lo, n_rows):
    rpw = n_rows // NW

    @functools.partial(
        pl.kernel,
        mesh=_sc_mesh,
        out_type=jax.ShapeDtypeStruct((n_rows * 16,), jnp.float32),
        scratch_types=[
            pltpu.VMEM((N,), jnp.float32),
            pltpu.VMEM((RG, N), jnp.float32),
            pltpu.VMEM((RG, N), jnp.float32),
            pltpu.VMEM((rpw * 16,), jnp.float32),
            pltpu.SemaphoreType.DMA,
            pltpu.SemaphoreType.DMA,
        ],
    )
    def sc_matvec(m_hbm, u_hbm, out_hbm, u_v, r0_v, r1_v, p_v, sem0, sem1):
        wid = lax.axis_index("s") * 2 + lax.axis_index("c")
        base = row_lo + wid * rpw
        lim = row_lo + n_rows - RG
        pltpu.sync_copy(u_hbm, u_v)

        def group_dot(rows_ref, out_base):
            def body(kk, accs):
                a = list(accs)
                for u in range(UNROLL):
                    j = kk * UNROLL + u
                    uv = u_v[pl.ds(j * 16, 16)]
                    for r in range(RG):
                        a[r] = a[r] + rows_ref[r, pl.ds(j * 16, 16)] * uv
                return tuple(a)

            z0 = tuple(jnp.zeros((16,), jnp.float32) for _ in range(RG))
            accs = lax.fori_loop(0, NCH // UNROLL, body, z0)
            for r in range(RG):
                p_v[pl.ds((out_base + r) * 16, 16)] = accs[r]

        pltpu.async_copy(m_hbm.at[pl.ds(base, RG)], r0_v, sem0)

        def pair(g, carry):
            r0 = base + 2 * RG * g
            pltpu.async_copy(m_hbm.at[pl.ds(r0 + RG, RG)], r1_v, sem1)
            pltpu.make_async_copy(m_hbm.at[pl.ds(r0, RG)], r0_v, sem0).wait()
            group_dot(r0_v, 2 * RG * g)
            nxt = jnp.minimum(r0 + 2 * RG, lim)
            pltpu.async_copy(m_hbm.at[pl.ds(nxt, RG)], r0_v, sem0)
            pltpu.make_async_copy(m_hbm.at[pl.ds(r0 + RG, RG)], r1_v,
                                  sem1).wait()
            group_dot(r1_v, 2 * RG * g + RG)
            return carry

        lax.fori_loop(0, rpw // (2 * RG), pair, 0)
        pltpu.make_async_copy(m_hbm.at[pl.ds(base, RG)], r0_v, sem0).wait()
        pltpu.sync_copy(p_v, out_hbm.at[pl.ds(wid * rpw * 16, rpw * 16)])

    return sc_matvec


_sc_pass2 = _make_sc_matvec(N_TC, R_SC)
_sc_readout = _make_sc_matvec(NK_TC, RK_SC)


def _pass2_body(adj_ref, v_ref, b2_ref, z_ref):
    z_ref[...] = (jnp.dot(adj_ref[...], v_ref[...],
                          preferred_element_type=jnp.float32)
                  + b2_ref[...])


def _zasm_body(ztc_ref, zp_ref, b2_ref, z_ref):
    z_ref[:N_TC, :] = ztc_ref[...]
    z_ref[N_TC:, :] = (jnp.sum(zp_ref[...], axis=1, keepdims=True)
                       + b2_ref[...])


def _readout_body(l1w_ref, z_ref, l1b_ref, l2w_ref, o_ref):
    k = pl.program_id(0)
    h = jnp.dot(l1w_ref[...], z_ref[...],
                preferred_element_type=jnp.float32)
    h = jnp.maximum(h + l1b_ref[...], 0.0)
    part = jnp.sum(h * l2w_ref[...]).reshape(1, 1)

    @pl.when(k == 0)
    def _():
        o_ref[...] = part

    @pl.when(k > 0)
    def _():
        o_ref[...] += part


def _final_body(fp_ref, l1b_ref, l2w_ref, otc_ref, l2b_ref, o_ref):
    d = jnp.sum(fp_ref[...], axis=1, keepdims=True)
    h = jnp.maximum(d + l1b_ref[...], 0.0)
    o_ref[...] = (jnp.sum(h * l2w_ref[...]).reshape(1, 1)
                  + otc_ref[...] + l2b_ref[...])


def kernel(x, adj, W1, b1, W2, b2, L1_W, L1_b, L2_W, L2_b):
    x2 = x[0]
    adj2 = adj[0]
    b1r = b1.reshape(1, NHID)
    b2r = b2.reshape(1, 1)
    l1b = L1_b.reshape(NH, 1)
    l2w = L2_W.reshape(NH, 1)
    l2b = L2_b.reshape(1, 1)

    v = pl.pallas_call(
        _pass1_body,
        grid=(NB1,),
        in_specs=[
            pl.BlockSpec((BM, N), lambda i: (i, 0)),
            pl.BlockSpec((N, NFEAT), lambda i: (0, 0)),
            pl.BlockSpec((NFEAT, NHID), lambda i: (0, 0)),
            pl.BlockSpec((1, NHID), lambda i: (0, 0)),
            pl.BlockSpec((NHID, 1), lambda i: (0, 0)),
        ],
        out_specs=pl.BlockSpec((BM, 1), lambda i: (i, 0)),
        out_shape=jax.ShapeDtypeStruct((N, 1), jnp.float32),
        scratch_shapes=[pltpu.VMEM((N, NHID), jnp.float32)],
    )(adj2, x2, W1, b1r, W2)

    zp = _sc_pass2(adj2, v.reshape(N))

    z_tc = pl.pallas_call(
        _pass2_body,
        grid=(NB2,),
        in_specs=[
            pl.BlockSpec((BM2, N), lambda i: (i, 0)),
            pl.BlockSpec((N, 1), lambda i: (0, 0)),
            pl.BlockSpec((1, 1), lambda i: (0, 0)),
        ],
        out_specs=pl.BlockSpec((BM2, 1), lambda i: (i, 0)),
        out_shape=jax.ShapeDtypeStruct((N_TC, 1), jnp.float32),
    )(adj2, v, b2r)

    z = pl.pallas_call(
        _zasm_body,
        in_specs=[
            pl.BlockSpec((N_TC, 1), lambda: (0, 0)),
            pl.BlockSpec((R_SC, 16), lambda: (0, 0)),
            pl.BlockSpec((1, 1), lambda: (0, 0)),
        ],
        out_specs=pl.BlockSpec((N, 1), lambda: (0, 0)),
        out_shape=jax.ShapeDtypeStruct((N, 1), jnp.float32),
    )(z_tc, zp.reshape(R_SC, 16), b2r)

    fp = _sc_readout(L1_W, z.reshape(N))

    out_tc = pl.pallas_call(
        _readout_body,
        grid=(NBK,),
        in_specs=[
            pl.BlockSpec((BK2, N), lambda k: (k, 0)),
            pl.BlockSpec((N, 1), lambda k: (0, 0)),
            pl.BlockSpec((BK2, 1), lambda k: (k, 0)),
            pl.BlockSpec((BK2, 1), lambda k: (k, 0)),
        ],
        out_specs=pl.BlockSpec((1, 1), lambda k: (0, 0)),
        out_shape=jax.ShapeDtypeStruct((1, 1), jnp.float32),
    )(L1_W, z, l1b, l2w)

    out = pl.pallas_call(
        _final_body,
        in_specs=[
            pl.BlockSpec((RK_SC, 16), lambda: (0, 0)),
            pl.BlockSpec((RK_SC, 1), lambda: (0, 0)),
            pl.BlockSpec((RK_SC, 1), lambda: (0, 0)),
            pl.BlockSpec((1, 1), lambda: (0, 0)),
            pl.BlockSpec((1, 1), lambda: (0, 0)),
        ],
        out_specs=pl.BlockSpec((1, 1), lambda: (0, 0)),
        out_shape=jax.ShapeDtypeStruct((1, 1), jnp.float32),
    )(fp.reshape(RK_SC, 16), l1b[NK_TC:], l2w[NK_TC:], out_tc, l2b)

    return out


# speedup vs baseline: 1.0701x; 1.0701x over previous
import functools

import jax
import jax.numpy as jnp
from jax import lax
from jax.experimental import pallas as pl
from jax.experimental.pallas import tpu as pltpu
from jax.experimental.pallas import tpu_sc as plsc

N = 10000
NFEAT = 128
NHID = 128
NH = N // 2

BM1 = 400
R_SC = 3000
W_P2 = 30
N_TC = N - R_SC
BM2 = 200
RK_SC = 1400
W_RD = 25
NK_TC = NH - RK_SC
BK2 = 400

NCH = N // 16
UNROLL = 25
RG = 4

NB1 = N // BM1
NB2 = N_TC // BM2
NBK = NK_TC // BK2


def _pass1_body(adj_ref, x_ref, w1_ref, b1_ref, w2_ref, v_ref, s1_ref):
    i = pl.program_id(0)

    @pl.when(i == 0)
    def _():
        s1_ref[...] = jnp.dot(x_ref[...], w1_ref[...],
                              preferred_element_type=jnp.float32)

    h = jnp.dot(adj_ref[...], s1_ref[...],
                preferred_element_type=jnp.float32)
    h = jnp.maximum(h + b1_ref[...], 0.0)
    v_ref[...] = jnp.dot(h, w2_ref[...],
                         preferred_element_type=jnp.float32)


_sc_mesh = plsc.VectorSubcoreMesh(core_axis_name="c", subcore_axis_name="s")


def _make_sc_matvec(row_lo, n_rows, workers):
    rpw = n_rows // workers
    groups = rpw // RG
    pairs = groups // 2
    tail = groups % 2

    @functools.partial(
        pl.kernel,
        mesh=_sc_mesh,
        out_type=jax.ShapeDtypeStruct((n_rows * 16,), jnp.float32),
        scratch_types=[
            pltpu.VMEM((N,), jnp.float32),
            pltpu.VMEM((RG, N), jnp.float32),
            pltpu.VMEM((RG, N), jnp.float32),
            pltpu.VMEM((rpw * 16,), jnp.float32),
            pltpu.SemaphoreType.DMA,
            pltpu.SemaphoreType.DMA,
        ],
    )
    def sc_matvec(m_hbm, u_hbm, out_hbm, u_v, r0_v, r1_v, p_v, sem0, sem1):
        wid = lax.axis_index("s") * 2 + lax.axis_index("c")

        @pl.when(wid < workers)
        def _():
            base = row_lo + wid * rpw
            lim = row_lo + n_rows - RG
            pltpu.sync_copy(u_hbm, u_v)

            def group_dot(rows_ref, out_base):
                def body(kk, accs):
                    a = list(accs)
                    for u in range(UNROLL):
                        j = kk * UNROLL + u
                        uv = u_v[pl.ds(j * 16, 16)]
                        for r in range(RG):
                            a[r] = a[r] + rows_ref[r, pl.ds(j * 16, 16)] * uv
                    return tuple(a)

                z0 = tuple(jnp.zeros((16,), jnp.float32) for _ in range(RG))
                accs = lax.fori_loop(0, NCH // UNROLL, body, z0)
                for r in range(RG):
                    p_v[pl.ds((out_base + r) * 16, 16)] = accs[r]

            pltpu.async_copy(m_hbm.at[pl.ds(base, RG)], r0_v, sem0)

            def pair(g, carry):
                r0 = base + 2 * RG * g
                pltpu.async_copy(m_hbm.at[pl.ds(r0 + RG, RG)], r1_v, sem1)
                pltpu.make_async_copy(m_hbm.at[pl.ds(r0, RG)], r0_v,
                                      sem0).wait()
                group_dot(r0_v, 2 * RG * g)
                nxt = jnp.minimum(r0 + 2 * RG, lim)
                pltpu.async_copy(m_hbm.at[pl.ds(nxt, RG)], r0_v, sem0)
                pltpu.make_async_copy(m_hbm.at[pl.ds(r0 + RG, RG)], r1_v,
                                      sem1).wait()
                group_dot(r1_v, 2 * RG * g + RG)
                return carry

            lax.fori_loop(0, pairs, pair, 0)

            if tail:
                rlast = base + (groups - 1) * RG
                pltpu.make_async_copy(m_hbm.at[pl.ds(rlast, RG)], r0_v,
                                      sem0).wait()
                group_dot(r0_v, (groups - 1) * RG)
            else:
                pltpu.make_async_copy(m_hbm.at[pl.ds(base, RG)], r0_v,
                                      sem0).wait()

            pltpu.sync_copy(p_v, out_hbm.at[pl.ds(wid * rpw * 16, rpw * 16)])

    return sc_matvec


_sc_pass2 = _make_sc_matvec(N_TC, R_SC, W_P2)
_sc_readout = _make_sc_matvec(NK_TC, RK_SC, W_RD)


def _pass2_body(adj_ref, v_ref, b2_ref, z_ref):
    z_ref[...] = (jnp.dot(adj_ref[...], v_ref[...],
                          preferred_element_type=jnp.float32)
                  + b2_ref[...])


def _zasm_body(ztc_ref, zp_ref, b2_ref, z_ref):
    z_ref[:N_TC, :] = ztc_ref[...]
    z_ref[N_TC:, :] = (jnp.sum(zp_ref[...], axis=1, keepdims=True)
                       + b2_ref[...])


def _readout_body(l1w_ref, z_ref, l1b_ref, l2w_ref, o_ref):
    k = pl.program_id(0)
    h = jnp.dot(l1w_ref[...], z_ref[...],
                preferred_element_type=jnp.float32)
    h = jnp.maximum(h + l1b_ref[...], 0.0)
    part = jnp.sum(h * l2w_ref[...]).reshape(1, 1)

    @pl.when(k == 0)
    def _():
        o_ref[...] = part

    @pl.when(k > 0)
    def _():
        o_ref[...] += part


def _final_body(fp_ref, l1b_ref, l2w_ref, otc_ref, l2b_ref, o_ref):
    d = jnp.sum(fp_ref[...], axis=1, keepdims=True)
    h = jnp.maximum(d + l1b_ref[...], 0.0)
    o_ref[...] = (jnp.sum(h * l2w_ref[...]).reshape(1, 1)
                  + otc_ref[...] + l2b_ref[...])


def kernel(x, adj, W1, b1, W2, b2, L1_W, L1_b, L2_W, L2_b):
    x2 = x[0]
    adj2 = adj[0]
    b1r = b1.reshape(1, NHID)
    b2r = b2.reshape(1, 1)
    l1b = L1_b.reshape(NH, 1)
    l2w = L2_W.reshape(NH, 1)
    l2b = L2_b.reshape(1, 1)

    v = pl.pallas_call(
        _pass1_body,
        grid=(NB1,),
        in_specs=[
            pl.BlockSpec((BM1, N), lambda i: (i, 0)),
            pl.BlockSpec((N, NFEAT), lambda i: (0, 0)),
            pl.BlockSpec((NFEAT, NHID), lambda i: (0, 0)),
            pl.BlockSpec((1, NHID), lambda i: (0, 0)),
            pl.BlockSpec((NHID, 1), lambda i: (0, 0)),
        ],
        out_specs=pl.BlockSpec((BM1, 1), lambda i: (i, 0)),
        out_shape=jax.ShapeDtypeStruct((N, 1), jnp.float32),
        scratch_shapes=[pltpu.VMEM((N, NHID), jnp.float32)],
    )(adj2, x2, W1, b1r, W2)

    zp = _sc_pass2(adj2, v.reshape(N))

    z_tc = pl.pallas_call(
        _pass2_body,
        grid=(NB2,),
        in_specs=[
            pl.BlockSpec((BM2, N), lambda i: (i, 0)),
            pl.BlockSpec((N, 1), lambda i: (0, 0)),
            pl.BlockSpec((1, 1), lambda i: (0, 0)),
        ],
        out_specs=pl.BlockSpec((BM2, 1), lambda i: (i, 0)),
        out_shape=jax.ShapeDtypeStruct((N_TC, 1), jnp.float32),
    )(adj2, v, b2r)

    z = pl.pallas_call(
        _zasm_body,
        in_specs=[
            pl.BlockSpec((N_TC, 1), lambda: (0, 0)),
            pl.BlockSpec((R_SC, 16), lambda: (0, 0)),
            pl.BlockSpec((1, 1), lambda: (0, 0)),
        ],
        out_specs=pl.BlockSpec((N, 1), lambda: (0, 0)),
        out_shape=jax.ShapeDtypeStruct((N, 1), jnp.float32),
    )(z_tc, zp.reshape(R_SC, 16), b2r)

    fp = _sc_readout(L1_W, z.reshape(N))

    out_tc = pl.pallas_call(
        _readout_body,
        grid=(NBK,),
        in_specs=[
            pl.BlockSpec((BK2, N), lambda k: (k, 0)),
            pl.BlockSpec((N, 1), lambda k: (0, 0)),
            pl.BlockSpec((BK2, 1), lambda k: (k, 0)),
            pl.BlockSpec((BK2, 1), lambda k: (k, 0)),
        ],
        out_specs=pl.BlockSpec((1, 1), lambda k: (0, 0)),
        out_shape=jax.ShapeDtypeStruct((1, 1), jnp.float32),
    )(L1_W, z, l1b, l2w)

    out = pl.pallas_call(
        _final_body,
        in_specs=[
            pl.BlockSpec((RK_SC, 16), lambda: (0, 0)),
            pl.BlockSpec((RK_SC, 1), lambda: (0, 0)),
            pl.BlockSpec((RK_SC, 1), lambda: (0, 0)),
            pl.BlockSpec((1, 1), lambda: (0, 0)),
            pl.BlockSpec((1, 1), lambda: (0, 0)),
        ],
        out_specs=pl.BlockSpec((1, 1), lambda: (0, 0)),
        out_shape=jax.ShapeDtypeStruct((1, 1), jnp.float32),
    )(fp.reshape(RK_SC, 16), l1b[NK_TC:], l2w[NK_TC:], out_tc, l2b)

    return out


# speedup vs baseline: 1.0832x; 1.0123x over previous
import functools

import jax
import jax.numpy as jnp
from jax import lax
from jax.experimental import pallas as pl
from jax.experimental.pallas import tpu as pltpu
from jax.experimental.pallas import tpu_sc as plsc

N = 10000
NFEAT = 128
NHID = 128
NH = N // 2

BM1 = 400
R_SC = 3840
W_P2 = 30
N_TC = N - R_SC
BM2 = 280
RK_SC = 1920
W_RD = 30
NK_TC = NH - RK_SC
BK2 = 280

NCH = N // 16
UNROLL = 25
RG = 4

NB1 = N // BM1
NB2 = N_TC // BM2
NBK = NK_TC // BK2

def _lane_iota():
    return lax.broadcasted_iota(jnp.int32, (16,), 0)


def _pass1_body(adj_ref, x_ref, w1_ref, b1_ref, w2_ref, v_ref, s1_ref):
    i = pl.program_id(0)

    @pl.when(i == 0)
    def _():
        s1_ref[...] = jnp.dot(x_ref[...], w1_ref[...],
                              preferred_element_type=jnp.float32)

    h = jnp.dot(adj_ref[...], s1_ref[...],
                preferred_element_type=jnp.float32)
    h = jnp.maximum(h + b1_ref[...], 0.0)
    v_ref[...] = jnp.dot(h, w2_ref[...],
                         preferred_element_type=jnp.float32)


_sc_mesh = plsc.VectorSubcoreMesh(core_axis_name="c", subcore_axis_name="s")


def _group_dots(rows_ref, u_v):
    def body(kk, accs):
        a = list(accs)
        for u in range(UNROLL):
            j = kk * UNROLL + u
            uv = u_v[pl.ds(j * 16, 16)]
            for r in range(RG):
                a[r] = a[r] + rows_ref[r, pl.ds(j * 16, 16)] * uv
        return tuple(a)

    z0 = tuple(jnp.zeros((16,), jnp.float32) for _ in range(RG))
    return lax.fori_loop(0, NCH // UNROLL, body, z0)


def _rsum_vec(vec):
    t = vec
    for sh in (8, 4, 2, 1):
        idx = jnp.bitwise_xor(_lane_iota(), sh)
        t = t + jnp.take(t, idx, axis=0)
    return t


@functools.partial(
    pl.kernel,
    mesh=_sc_mesh,
    out_type=jax.ShapeDtypeStruct((R_SC,), jnp.float32),
    scratch_types=[
        pltpu.VMEM((N,), jnp.float32),
        pltpu.VMEM((16,), jnp.float32),
        pltpu.VMEM((RG, N), jnp.float32),
        pltpu.VMEM((RG, N), jnp.float32),
        pltpu.VMEM((R_SC // W_P2,), jnp.float32),
        pltpu.SemaphoreType.DMA,
        pltpu.SemaphoreType.DMA,
    ],
)
def _sc_pass2(m_hbm, u_hbm, b2_hbm, out_hbm, u_v, b2_v, r0_v, r1_v, p_v,
              sem0, sem1):
    rpw = R_SC // W_P2
    wid = lax.axis_index("s") * 2 + lax.axis_index("c")

    @pl.when(wid < W_P2)
    def _():
        base = N_TC + wid * rpw
        pltpu.sync_copy(u_hbm, u_v)
        pltpu.sync_copy(b2_hbm, b2_v)
        b2s = b2_v[...][0]

        pltpu.async_copy(m_hbm.at[pl.ds(base, RG)], r0_v, sem0)
        pltpu.async_copy(m_hbm.at[pl.ds(base + RG, RG)], r1_v, sem1)
        bufs = (r0_v, r1_v)
        sems = (sem0, sem1)
        lim = N - RG

        def window(w, _):
            vec = jnp.zeros((16,), jnp.float32)
            for g in range(4):
                gg = w * 4 + g
                buf, sem = bufs[g % 2], sems[g % 2]
                r0 = base + gg * RG
                pltpu.make_async_copy(m_hbm.at[pl.ds(r0, RG)], buf,
                                      sem).wait()
                accs = _group_dots(buf, u_v)
                for r in range(RG):
                    vec = jnp.where(_lane_iota() == 4 * g + r,
                                    _rsum_vec(accs[r]) + b2s, vec)
                nxt = jnp.minimum(r0 + 2 * RG, lim)
                pltpu.async_copy(m_hbm.at[pl.ds(nxt, RG)], buf, sem)
            p_v[pl.ds(w * 16, 16)] = vec
            return 0

        lax.fori_loop(0, rpw // 16, window, 0)
        pltpu.make_async_copy(m_hbm.at[pl.ds(base, RG)], r0_v, sem0).wait()
        pltpu.make_async_copy(m_hbm.at[pl.ds(base, RG)], r1_v, sem1).wait()
        pltpu.sync_copy(p_v, out_hbm.at[pl.ds(wid * rpw, rpw)])


@functools.partial(
    pl.kernel,
    mesh=_sc_mesh,
    out_type=jax.ShapeDtypeStruct((32 * 16,), jnp.float32),
    scratch_types=[
        pltpu.VMEM((N,), jnp.float32),
        pltpu.VMEM((RK_SC // W_RD,), jnp.float32),
        pltpu.VMEM((RK_SC // W_RD,), jnp.float32),
        pltpu.VMEM((RG, N), jnp.float32),
        pltpu.VMEM((RG, N), jnp.float32),
        pltpu.VMEM((16,), jnp.float32),
        pltpu.SemaphoreType.DMA,
        pltpu.SemaphoreType.DMA,
    ],
)
def _sc_readout(m_hbm, ztc_hbm, zbot_hbm, l1b_hbm, l2w_hbm, out_hbm,
                u_v, l1b_v, l2w_v, r0_v, r1_v, p_v, sem0, sem1):
    rpw = RK_SC // W_RD
    wid = lax.axis_index("s") * 2 + lax.axis_index("c")

    p_v[...] = jnp.zeros((16,), jnp.float32)

    @pl.when(wid < W_RD)
    def _():
        base = NK_TC + wid * rpw
        pltpu.sync_copy(ztc_hbm, u_v.at[pl.ds(0, N_TC)])
        pltpu.sync_copy(zbot_hbm, u_v.at[pl.ds(N_TC, R_SC)])
        pltpu.sync_copy(l1b_hbm.at[pl.ds(base, rpw)], l1b_v)
        pltpu.sync_copy(l2w_hbm.at[pl.ds(base, rpw)], l2w_v)

        pltpu.async_copy(m_hbm.at[pl.ds(base, RG)], r0_v, sem0)
        pltpu.async_copy(m_hbm.at[pl.ds(base + RG, RG)], r1_v, sem1)
        bufs = (r0_v, r1_v)
        sems = (sem0, sem1)
        lim = NH - RG

        def window(w, acc):
            for g in range(4):
                gg = w * 4 + g
                buf, sem = bufs[g % 2], sems[g % 2]
                r0 = base + gg * RG
                pltpu.make_async_copy(m_hbm.at[pl.ds(r0, RG)], buf,
                                      sem).wait()
                accs = _group_dots(buf, u_v)
                l1b_win = l1b_v[pl.ds(w * 16, 16)]
                l2w_win = l2w_v[pl.ds(w * 16, 16)]
                for r in range(RG):
                    li = 4 * g + r
                    h3 = jnp.maximum(_rsum_vec(accs[r]) + l1b_win[li], 0.0)
                    acc = jnp.where(_lane_iota() == li,
                                    acc + h3 * l2w_win[li], acc)
                nxt = jnp.minimum(r0 + 2 * RG, lim)
                pltpu.async_copy(m_hbm.at[pl.ds(nxt, RG)], buf, sem)
            return acc

        acc = lax.fori_loop(0, rpw // 16, window,
                            jnp.zeros((16,), jnp.float32))
        pltpu.make_async_copy(m_hbm.at[pl.ds(base, RG)], r0_v, sem0).wait()
        pltpu.make_async_copy(m_hbm.at[pl.ds(base, RG)], r1_v, sem1).wait()
        p_v[...] = acc

    pltpu.sync_copy(p_v, out_hbm.at[pl.ds(wid * 16, 16)])


def _pass2_body(adj_ref, v_ref, b2_ref, z_ref):
    z_ref[...] = (jnp.dot(adj_ref[...], v_ref[...],
                          preferred_element_type=jnp.float32)
                  + b2_ref[...])


def _readout_body(l1w_ref, ztc_ref, zbot_ref, l1b_ref, l2w_ref, o_ref,
                  z_ref):
    k = pl.program_id(0)

    @pl.when(k == 0)
    def _():
        z_ref[:N_TC, :] = ztc_ref[...]
        z_ref[N_TC:, :] = zbot_ref[...]

    h = jnp.dot(l1w_ref[...], z_ref[...],
                preferred_element_type=jnp.float32)
    h = jnp.maximum(h + l1b_ref[...], 0.0)
    part = jnp.sum(h * l2w_ref[...]).reshape(1, 1)

    @pl.when(k == 0)
    def _():
        o_ref[...] = part

    @pl.when(k > 0)
    def _():
        o_ref[...] += part


def _final_body(fw_ref, otc_ref, l2b_ref, o_ref):
    o_ref[...] = (jnp.sum(fw_ref[...]).reshape(1, 1)
                  + otc_ref[...] + l2b_ref[...])


def kernel(x, adj, W1, b1, W2, b2, L1_W, L1_b, L2_W, L2_b):
    x2 = x[0]
    adj2 = adj[0]
    b1r = b1.reshape(1, NHID)
    b2r = b2.reshape(1, 1)
    l1b = L1_b.reshape(NH, 1)
    l2w = L2_W.reshape(NH, 1)
    l2b = L2_b.reshape(1, 1)

    v = pl.pallas_call(
        _pass1_body,
        grid=(NB1,),
        in_specs=[
            pl.BlockSpec((BM1, N), lambda i: (i, 0)),
            pl.BlockSpec((N, NFEAT), lambda i: (0, 0)),
            pl.BlockSpec((NFEAT, NHID), lambda i: (0, 0)),
            pl.BlockSpec((1, NHID), lambda i: (0, 0)),
            pl.BlockSpec((NHID, 1), lambda i: (0, 0)),
        ],
        out_specs=pl.BlockSpec((BM1, 1), lambda i: (i, 0)),
        out_shape=jax.ShapeDtypeStruct((N, 1), jnp.float32),
        scratch_shapes=[pltpu.VMEM((N, NHID), jnp.float32)],
    )(adj2, x2, W1, b1r, W2)

    zbot = _sc_pass2(adj2, v.reshape(N),
                     jnp.broadcast_to(b2, (16,)))

    z_tc = pl.pallas_call(
        _pass2_body,
        grid=(NB2,),
        in_specs=[
            pl.BlockSpec((BM2, N), lambda i: (i, 0)),
            pl.BlockSpec((N, 1), lambda i: (0, 0)),
            pl.BlockSpec((1, 1), lambda i: (0, 0)),
        ],
        out_specs=pl.BlockSpec((BM2, 1), lambda i: (i, 0)),
        out_shape=jax.ShapeDtypeStruct((N_TC, 1), jnp.float32),
    )(adj2, v, b2r)

    fw = _sc_readout(L1_W, z_tc.reshape(N_TC), zbot, L1_b,
                     L2_W.reshape(NH))

    out_tc = pl.pallas_call(
        _readout_body,
        grid=(NBK,),
        in_specs=[
            pl.BlockSpec((BK2, N), lambda k: (k, 0)),
            pl.BlockSpec((N_TC, 1), lambda k: (0, 0)),
            pl.BlockSpec((R_SC, 1), lambda k: (0, 0)),
            pl.BlockSpec((BK2, 1), lambda k: (k, 0)),
            pl.BlockSpec((BK2, 1), lambda k: (k, 0)),
        ],
        out_specs=pl.BlockSpec((1, 1), lambda k: (0, 0)),
        out_shape=jax.ShapeDtypeStruct((1, 1), jnp.float32),
        scratch_shapes=[pltpu.VMEM((N, 1), jnp.float32)],
    )(L1_W, z_tc, zbot.reshape(R_SC, 1), l1b, l2w)

    out = pl.pallas_call(
        _final_body,
        in_specs=[
            pl.BlockSpec((32, 16), lambda: (0, 0)),
            pl.BlockSpec((1, 1), lambda: (0, 0)),
            pl.BlockSpec((1, 1), lambda: (0, 0)),
        ],
        out_specs=pl.BlockSpec((1, 1), lambda: (0, 0)),
        out_shape=jax.ShapeDtypeStruct((1, 1), jnp.float32),
    )(fw.reshape(32, 16), out_tc, l2b)

    return out


# speedup vs baseline: 1.2566x; 1.1601x over previous
import jax
import jax.numpy as jnp
from jax.experimental import pallas as pl
from jax.experimental.pallas import tpu as pltpu

N = 10000
NFEAT = 128
NHID = 128
NH = N // 2

BM = 200
BK = 200
NB = N // BM
NBK = NH // BK


def _mega_body(adj_ref, x_ref, w1_ref, b1_ref, w2_ref,
               l1w_ref, l1b_ref, l2w_ref, b2_ref, l2b_ref,
               o_ref, s1_ref, v_ref, vr_ref, z_ref, zr_ref):
    i = pl.program_id(0)

    @pl.when(i == 0)
    def _():
        s1_ref[...] = jnp.dot(x_ref[...], w1_ref[...],
                              preferred_element_type=jnp.float32)

    @pl.when(i < NB)
    def _():
        h = jnp.dot(adj_ref[...], s1_ref[...],
                    preferred_element_type=jnp.float32)
        h = jnp.maximum(h + b1_ref[...], 0.0)
        vb = jnp.sum(h * w2_ref[...], axis=1, keepdims=True)
        v_ref[pl.ds(i * BM, BM), :] = vb

    @pl.when(i == NB)
    def _():
        vr_ref[...] = v_ref[...].T

    @pl.when((i >= NB) & (i < 2 * NB))
    def _():
        j = i - NB
        zb = jnp.sum(adj_ref[...] * vr_ref[...], axis=1,
                     keepdims=True)
        z_ref[pl.ds(j * BM, BM), :] = zb + b2_ref[...]

    @pl.when(i == 2 * NB)
    def _():
        zr_ref[...] = z_ref[...].T

    @pl.when(i >= 2 * NB)
    def _():
        k = i - 2 * NB
        d = jnp.sum(l1w_ref[...] * zr_ref[...], axis=1,
                    keepdims=True)
        h3 = jnp.maximum(d + l1b_ref[...], 0.0)
        part = jnp.sum(h3 * l2w_ref[...]).reshape(1, 1)

        @pl.when(k == 0)
        def _():
            o_ref[...] = part + l2b_ref[...]

        @pl.when(k > 0)
        def _():
            o_ref[...] += part


def _adj_row(i):
    return (jnp.where(i < NB, i, jnp.where(i < 2 * NB, i - NB, NB - 1)), 0)


def _l1_row(i):
    return (jnp.clip(i - 2 * NB, 0, NBK - 1), 0)


def kernel(x, adj, W1, b1, W2, b2, L1_W, L1_b, L2_W, L2_b):
    x2 = x[0]
    adj2 = adj[0]
    b1r = b1.reshape(1, NHID)
    w2r = W2.reshape(1, NHID)
    b2r = b2.reshape(1, 1)
    l1b = L1_b.reshape(NH, 1)
    l2w = L2_W.reshape(NH, 1)
    l2b = L2_b.reshape(1, 1)

    out = pl.pallas_call(
        _mega_body,
        grid=(2 * NB + NBK,),
        in_specs=[
            pl.BlockSpec((BM, N), _adj_row),
            pl.BlockSpec((N, NFEAT), lambda i: (0, 0)),
            pl.BlockSpec((NFEAT, NHID), lambda i: (0, 0)),
            pl.BlockSpec((1, NHID), lambda i: (0, 0)),
            pl.BlockSpec((1, NHID), lambda i: (0, 0)),
            pl.BlockSpec((BK, N), _l1_row),
            pl.BlockSpec((BK, 1), _l1_row),
            pl.BlockSpec((BK, 1), _l1_row),
            pl.BlockSpec((1, 1), lambda i: (0, 0)),
            pl.BlockSpec((1, 1), lambda i: (0, 0)),
        ],
        out_specs=pl.BlockSpec((1, 1), lambda i: (0, 0)),
        out_shape=jax.ShapeDtypeStruct((1, 1), jnp.float32),
        scratch_shapes=[
            pltpu.VMEM((N, NHID), jnp.float32),
            pltpu.VMEM((N, 1), jnp.float32),
            pltpu.VMEM((1, N), jnp.float32),
            pltpu.VMEM((N, 1), jnp.float32),
            pltpu.VMEM((1, N), jnp.float32),
        ],
    )(adj2, x2, W1, b1r, w2r, L1_W, l1b, l2w, b2r, l2b)

    return out
